# baseline (speedup 1.0000x reference)
import functools

import jax
import jax.numpy as jnp
from jax import lax
from jax.experimental import pallas as pl
from jax.experimental.pallas import tpu as pltpu
from jax.experimental.pallas import tpu_sc as plsc

_B, _T = 16384, 4096
_NC, _NS = 2, 16
_NW = _NC * _NS
_ROWS_PER_W = _B // _NW
_L = 16

_CHUNK_ROWS = 8
_NBUF = 2
_NCHUNKS = _ROWS_PER_W // _CHUNK_ROWS


@functools.partial(
    pl.kernel,
    out_type=jax.ShapeDtypeStruct((_B, _T), jnp.float32),
    mesh=plsc.VectorSubcoreMesh(core_axis_name="c", subcore_axis_name="s"),
    scratch_types=[
        pltpu.VMEM((_NBUF, _CHUNK_ROWS, _T), jnp.float32),
        pltpu.SemaphoreType.DMA((_NBUF,)),
        pltpu.SemaphoreType.DMA((_NBUF,)),
    ],
)
def _sc_copy_swap(x_ref, o_ref, bufs, in_sem, out_sem):
    wid = lax.axis_index("s") * _NC + lax.axis_index("c")
    base = wid * _ROWS_PER_W
    lane = lax.iota(jnp.int32, _L)
    perm = jnp.where(lane == 0, 1, jnp.where(lane == 1, 0, lane))
    dnums = lax.GatherDimensionNumbers(
        offset_dims=(), collapsed_slice_dims=(0,), start_index_map=(0,))

    def in_copy(c, b):
        return pltpu.make_async_copy(
            x_ref.at[pl.ds(base + c * _CHUNK_ROWS, _CHUNK_ROWS), pl.ds(0, _T)],
            bufs.at[b],
            in_sem.at[b])

    def out_copy(c, b):
        return pltpu.make_async_copy(
            bufs.at[b],
            o_ref.at[pl.ds(base + c * _CHUNK_ROWS, _CHUNK_ROWS), pl.ds(0, _T)],
            out_sem.at[b])

    for b in range(_NBUF):
        in_copy(b, b).start()

    def round_body(g, carry):
        for b in range(_NBUF):
            c = g * _NBUF + b
            in_copy(c, b).wait()
            for r in range(_CHUNK_ROWS):
                v = bufs[b, r, pl.ds(0, _L)]
                swapped = lax.gather(
                    v, perm[:, None], dnums, (1,),
                    mode=lax.GatherScatterMode.PROMISE_IN_BOUNDS)
                bufs[b, r, pl.ds(0, _L)] = swapped
            out_copy(c, b).start()
            nxt = c + _NBUF

            @pl.when(nxt < _NCHUNKS)
            def _prefetch():
                out_copy(c, b).wait()
                in_copy(nxt, b).start()

        return carry

    lax.fori_loop(0, _NCHUNKS // _NBUF, round_body, 0)
    for b in range(_NBUF):
        out_copy(_NCHUNKS - _NBUF + b, b).wait()


def kernel(tokens):
    return _sc_copy_swap(tokens)


# speedup vs baseline: 1.0042x; 1.0042x over previous
import functools

import jax
import jax.numpy as jnp
from jax import lax
from jax.experimental import pallas as pl
from jax.experimental.pallas import tpu as pltpu
from jax.experimental.pallas import tpu_sc as plsc

_B, _T = 16384, 4096
_NC, _NS = 2, 16
_NW = _NC * _NS
_ROWS_PER_W = _B // _NW
_L = 16

_CHUNK_ROWS = 8
_NBUF = 3
_NCHUNKS = _ROWS_PER_W // _CHUNK_ROWS
_MAIN_ROUNDS = _NCHUNKS // _NBUF


@functools.partial(
    pl.kernel,
    out_type=jax.ShapeDtypeStruct((_B, _T), jnp.float32),
    mesh=plsc.VectorSubcoreMesh(core_axis_name="c", subcore_axis_name="s"),
    scratch_types=[
        pltpu.VMEM((_NBUF, _CHUNK_ROWS, _T), jnp.float32),
        pltpu.SemaphoreType.DMA((_NBUF,)),
        pltpu.SemaphoreType.DMA((_NBUF,)),
    ],
)
def _sc_copy_swap(x_ref, o_ref, bufs, in_sem, out_sem):
    wid = lax.axis_index("s") * _NC + lax.axis_index("c")
    base = wid * _ROWS_PER_W
    lane = lax.iota(jnp.int32, _L)
    perm = jnp.where(lane == 0, 1, jnp.where(lane == 1, 0, lane))
    dnums = lax.GatherDimensionNumbers(
        offset_dims=(), collapsed_slice_dims=(0,), start_index_map=(0,))

    def in_copy(c, b):
        return pltpu.make_async_copy(
            x_ref.at[pl.ds(base + c * _CHUNK_ROWS, _CHUNK_ROWS), pl.ds(0, _T)],
            bufs.at[b],
            in_sem.at[b])

    def out_copy(c, b):
        return pltpu.make_async_copy(
            bufs.at[b],
            o_ref.at[pl.ds(base + c * _CHUNK_ROWS, _CHUNK_ROWS), pl.ds(0, _T)],
            out_sem.at[b])

    def process(c, b):
        in_copy(c, b).wait()
        for r in range(_CHUNK_ROWS):
            v = bufs[b, r, pl.ds(0, _L)]
            swapped = lax.gather(
                v, perm[:, None], dnums, (1,),
                mode=lax.GatherScatterMode.PROMISE_IN_BOUNDS)
            bufs[b, r, pl.ds(0, _L)] = swapped
        out_copy(c, b).start()
        nxt = c + _NBUF

        @pl.when(nxt < _NCHUNKS)
        def _prefetch():
            out_copy(c, b).wait()
            in_copy(nxt, b).start()

    for b in range(_NBUF):
        in_copy(b, b).start()

    def round_body(g, carry):
        for b in range(_NBUF):
            process(g * _NBUF + b, b)
        return carry

    lax.fori_loop(0, _MAIN_ROUNDS, round_body, 0)
    for c in range(_MAIN_ROUNDS * _NBUF, _NCHUNKS):
        process(c, c % _NBUF)
    for b in range(_NBUF):
        last = max(c for c in range(_NCHUNKS) if c % _NBUF == b)
        out_copy(last, b).wait()


def kernel(tokens):
    return _sc_copy_swap(tokens)


# speedup vs baseline: 1.0803x; 1.0758x over previous
import functools

import jax
import jax.numpy as jnp
from jax import lax
from jax.experimental import pallas as pl
from jax.experimental.pallas import tpu as pltpu
from jax.experimental.pallas import tpu_sc as plsc

_B, _T = 16384, 4096
_BLOCK_ROWS = 512
_NC, _NS = 2, 16
_NW = _NC * _NS
_ROWS_PER_W = _B // _NW
_SLAB = 128
_L = 16
_UNROLL = 8


@functools.partial(
    pl.kernel,
    mesh=plsc.VectorSubcoreMesh(core_axis_name="c", subcore_axis_name="s"),
    scratch_types=[
        pltpu.VMEM((_ROWS_PER_W, _SLAB), jnp.float32),
    ],
)
def _sc_swap(y_ref, blk):
    wid = lax.axis_index("s") * _NC + lax.axis_index("c")
    base = wid * _ROWS_PER_W
    pltpu.sync_copy(y_ref.at[pl.ds(base, _ROWS_PER_W), pl.ds(0, _SLAB)], blk)
    lane = lax.iota(jnp.int32, _L)
    perm = jnp.where(lane == 0, 1, jnp.where(lane == 1, 0, lane))
    dnums = lax.GatherDimensionNumbers(
        offset_dims=(), collapsed_slice_dims=(0,), start_index_map=(0,))

    def body(i, carry):
        for u in range(_UNROLL):
            r = i * _UNROLL + u
            v = blk[r, pl.ds(0, _L)]
            swapped = lax.gather(
                v, perm[:, None], dnums, (1,),
                mode=lax.GatherScatterMode.PROMISE_IN_BOUNDS)
            blk[r, pl.ds(0, _L)] = swapped
        return carry

    lax.fori_loop(0, _ROWS_PER_W // _UNROLL, body, 0)
    pltpu.sync_copy(blk, y_ref.at[pl.ds(base, _ROWS_PER_W), pl.ds(0, _SLAB)])


def _copy_body(x_ref, o_ref):
    o_ref[...] = x_ref[...]


def _tc_copy(tokens):
    return pl.pallas_call(
        _copy_body,
        grid=(_B // _BLOCK_ROWS,),
        in_specs=[pl.BlockSpec((_BLOCK_ROWS, _T), lambda i: (i, 0))],
        out_specs=pl.BlockSpec((_BLOCK_ROWS, _T), lambda i: (i, 0)),
        out_shape=jax.ShapeDtypeStruct((_B, _T), tokens.dtype),
    )(tokens)


def kernel(tokens):
    y_ref = jax.new_ref(_tc_copy(tokens))
    _sc_swap(y_ref)
    return jax.freeze(y_ref)
